# speedup vs baseline: 21.0999x; 21.0999x over previous
import functools

import jax
import jax.numpy as jnp
from jax import lax
from jax.experimental import pallas as pl
from jax.experimental.pallas import tpu as pltpu

_D = 4
_NP = 8
_NEDGE = (1 << _D) - 1


def _thresh_body(rank, x_ref, m_ref, t_ref, cnt_ref, p_ref):
    p = pl.program_id(0)

    @pl.when(pl.program_id(1) == 0)
    def _decide_and_reset():
        acc = jnp.int32(0)
        for j in range(_NEDGE):
            acc += jnp.where(cnt_ref[j] >= rank, 1, 0).astype(jnp.int32)
        new_p = jnp.where(p == 0, 0, p_ref[0] * (1 << _D) + acc)
        p_ref[0] = new_p
        for j in range(_NEDGE):
            cnt_ref[j] = 0

        @pl.when(p == _NP)
        def _emit():
            t_ref[0, 0] = jnp.maximum(new_p, 1)

    @pl.when(p < _NP)
    def _accumulate():
        x = x_ref[...]
        m = m_ref[...]
        u = lax.bitcast_convert_type(x, jnp.int32)
        ueff = jnp.where((m > 0.0) & (x > 0.0), u, 0)
        shift = (_NP - 1 - p) * _D
        v = lax.shift_right_logical(ueff, shift)
        base = p_ref[0] * (1 << _D)
        for j in range(_NEDGE):
            cnt = jnp.sum((v >= base + (j + 1)).astype(jnp.int32))
            cnt_ref[j] = cnt_ref[j] + cnt


def _apply_body(t_ref, x_ref, m_ref, o_ref):
    x = x_ref[...]
    u = lax.bitcast_convert_type(x, jnp.int32)
    keep = (m_ref[...] > 0.0) & (x > 0.0) & (u >= t_ref[0, 0])
    o_ref[...] = jnp.where(keep, x, 0.0)


def kernel(x, token_mask):
    b, t, f = x.shape
    rows = b * t
    xf = x.reshape(rows, f)
    mf = token_mask.reshape(rows, 1).astype(jnp.float32)

    blk_r = 256 if rows % 256 == 0 else rows
    nb = rows // blk_r

    tval = pl.pallas_call(
        functools.partial(_thresh_body, 32 * rows),
        grid=(_NP + 1, nb),
        in_specs=[
            pl.BlockSpec((blk_r, f), lambda p, i: (jnp.where(p == _NP, 0, i), 0)),
            pl.BlockSpec((blk_r, 1), lambda p, i: (jnp.where(p == _NP, 0, i), 0)),
        ],
        out_specs=pl.BlockSpec(memory_space=pltpu.SMEM),
        out_shape=jax.ShapeDtypeStruct((1, 1), jnp.int32),
        scratch_shapes=[
            pltpu.SMEM((_NEDGE,), jnp.int32),
            pltpu.SMEM((1,), jnp.int32),
        ],
    )(xf, mf)

    blk_a = 512 if rows % 512 == 0 else blk_r
    out = pl.pallas_call(
        _apply_body,
        grid=(rows // blk_a,),
        in_specs=[
            pl.BlockSpec(memory_space=pltpu.SMEM),
            pl.BlockSpec((blk_a, f), lambda i: (i, 0)),
            pl.BlockSpec((blk_a, 1), lambda i: (i, 0)),
        ],
        out_specs=pl.BlockSpec((blk_a, f), lambda i: (i, 0)),
        out_shape=jax.ShapeDtypeStruct((rows, f), x.dtype),
    )(tval, xf, mf)

    return out.reshape(x.shape)
